# baseline (speedup 1.0000x reference)
import functools

import jax
import jax.numpy as jnp
from jax import lax
from jax.experimental import pallas as pl
from jax.experimental.pallas import tpu as pltpu
from jax.experimental.pallas import tpu_sc as plsc

N = 10000
E = 320000
D = 128
DE = 16
H = 128

NC = 2
NS = 16
CHUNK = 128
CHUNKS_PER_TILE = 80
E_PAD = NC * NS * CHUNKS_PER_TILE * CHUNK
N_PAD = 10112
ROWS_PER_TILE = N_PAD // NS
NIB = 4


def _sc_message_pass_body(h_hbm, e_hbm, sd_hbm, zeros_hbm,
                          out_hbm, ib, rows_v, e_v, agg_sh,
                          gsem, esem, isem):
    c = lax.axis_index("c")
    s = lax.axis_index("s")
    wid = c * NS + s
    base = s * ROWS_PER_TILE

    pltpu.sync_copy(zeros_hbm.at[pl.ds(base, ROWS_PER_TILE)],
                    agg_sh.at[pl.ds(base, ROWS_PER_TILE)])
    plsc.subcore_barrier()

    def fetch_idx_async(k, j):
        pltpu.async_copy(sd_hbm.at[0, wid, k], ib[j].at[0], isem[j])
        pltpu.async_copy(sd_hbm.at[1, wid, k], ib[j].at[1], isem[j])

    def fetch_gather(ij, b):
        pltpu.async_copy(h_hbm.at[ib[ij].at[0]], rows_v[b], gsem[b])

    def fetch_e(k):
        eoff = (wid * CHUNKS_PER_TILE + k) * CHUNK
        pltpu.async_copy(e_hbm.at[pl.ds(eoff, CHUNK)], e_v, esem)

    pltpu.sync_copy(sd_hbm.at[0, wid, 0], ib[0].at[0])
    pltpu.sync_copy(sd_hbm.at[1, wid, 0], ib[0].at[1])
    fetch_idx_async(1, 1)
    fetch_gather(0, 0)
    fetch_e(0)

    def step(k, j):
        b = j % 2
        nb = (b + 1) % 2
        pltpu.make_async_copy(h_hbm.at[ib[0].at[0]], rows_v[b],
                              gsem[b]).wait()
        pltpu.make_async_copy(e_hbm.at[pl.ds(0, CHUNK)], e_v, esem).wait()

        def row_body(r, carry2):
            for rr in range(2):
                for jj in range(H // 16):
                    sl = pl.ds(jj * 16, 16)
                    rows_v[b][2 * r + rr, sl] = jnp.maximum(
                        rows_v[b][2 * r + rr, sl] + e_v[2 * r + rr, sl], 0.0)
            return carry2

        lax.fori_loop(0, CHUNK // 2, row_body, 0, unroll=False)

        @pl.when(k + 1 < CHUNKS_PER_TILE)
        def _():
            @pl.when(k + 2 < CHUNKS_PER_TILE)
            def _():
                fetch_idx_async(k + 2, (j + 2) % NIB)

            fetch_e(k + 1)
            pltpu.make_async_copy(sd_hbm.at[0, wid, 0], ib[(j + 1) % NIB].at[0],
                                  isem[(j + 1) % NIB]).wait()
            pltpu.make_async_copy(sd_hbm.at[1, wid, 0], ib[(j + 1) % NIB].at[1],
                                  isem[(j + 1) % NIB]).wait()
            fetch_gather((j + 1) % NIB, nb)

        pltpu.sync_copy(rows_v[b], agg_sh.at[ib[j].at[1]], add=True)
        return None

    def outer_body(kk, carry):
        for j in range(NIB):
            step(kk * NIB + j, j)
        return carry

    lax.fori_loop(0, CHUNKS_PER_TILE // NIB, outer_body, 0, unroll=False)

    plsc.subcore_barrier()
    pltpu.sync_copy(agg_sh.at[pl.ds(base, ROWS_PER_TILE)],
                    out_hbm.at[c, pl.ds(base, ROWS_PER_TILE)])


def _sc_message_pass(h, e, sd_t, zeros):
    mesh = plsc.VectorSubcoreMesh(core_axis_name="c", subcore_axis_name="s")
    fn = pl.kernel(
        _sc_message_pass_body,
        out_type=jax.ShapeDtypeStruct((NC, N_PAD, H), jnp.float32),
        mesh=mesh,
        scratch_types=[
            [pltpu.VMEM((2, CHUNK), jnp.int32)] * NIB,
            [pltpu.VMEM((CHUNK, H), jnp.float32)] * 2,
            pltpu.VMEM((CHUNK, H), jnp.float32),
            pltpu.VMEM_SHARED((N_PAD, H), jnp.float32),
            [pltpu.SemaphoreType.DMA] * 2,
            pltpu.SemaphoreType.DMA,
            [pltpu.SemaphoreType.DMA] * NIB,
        ],
    )
    return fn(h, e, sd_t, zeros)


def _edge_mlp_body(ea_ref, w_ref, b_ref, o_ref):
    o_ref[...] = jnp.dot(ea_ref[...], w_ref[...],
                         preferred_element_type=jnp.float32) + b_ref[...]


def _edge_mlp(edge_attr, w_e, b_e):
    BE = 4000
    grid = (E // BE,)
    return pl.pallas_call(
        _edge_mlp_body,
        grid=grid,
        in_specs=[
            pl.BlockSpec((BE, DE), lambda i: (i, 0)),
            pl.BlockSpec((DE, H), lambda i: (0, 0)),
            pl.BlockSpec((1, H), lambda i: (0, 0)),
        ],
        out_specs=pl.BlockSpec((BE, H), lambda i: (i, 0)),
        out_shape=jax.ShapeDtypeStruct((E_PAD, H), jnp.float32),
    )(edge_attr, w_e, b_e)


def _node_mlp_body(h_ref, part_ref, w1_ref, b1_ref, w2_ref, b2_ref,
                   lng_ref, lnb_ref, eps_ref, o_ref, *, residual):
    h = h_ref[...]
    agg = part_ref[0] + part_ref[1]
    z = (1.0 + eps_ref[0]) * h + agg
    z1 = jnp.dot(z, w1_ref[...], preferred_element_type=jnp.float32)
    z1 = jnp.maximum(z1 + b1_ref[...], 0.0)
    z2 = jnp.dot(z1, w2_ref[...], preferred_element_type=jnp.float32)
    z2 = z2 + b2_ref[...]
    mu = jnp.mean(z2, axis=-1, keepdims=True)
    var = jnp.mean((z2 - mu) ** 2, axis=-1, keepdims=True)
    zn = (z2 - mu) * lax.rsqrt(var + 1e-5) * lng_ref[...] + lnb_ref[...]
    zr = jnp.maximum(zn, 0.0)
    if residual:
        o_ref[...] = h + 0.3 * zr
    else:
        o_ref[...] = zr


def _node_mlp(h, part, w1, b1, w2, b2, lng, lnb, eps, residual):
    BN = 1000
    grid = (N // BN,)
    body = functools.partial(_node_mlp_body, residual=residual)
    return pl.pallas_call(
        body,
        grid=grid,
        in_specs=[
            pl.BlockSpec((BN, H), lambda i: (i, 0)),
            pl.BlockSpec((NC, BN, H), lambda i: (0, i, 0)),
            pl.BlockSpec((H, 2 * H), lambda i: (0, 0)),
            pl.BlockSpec((1, 2 * H), lambda i: (0, 0)),
            pl.BlockSpec((2 * H, H), lambda i: (0, 0)),
            pl.BlockSpec((1, H), lambda i: (0, 0)),
            pl.BlockSpec((1, H), lambda i: (0, 0)),
            pl.BlockSpec((1, H), lambda i: (0, 0)),
            pl.BlockSpec(memory_space=pltpu.SMEM),
        ],
        out_specs=pl.BlockSpec((BN, H), lambda i: (i, 0)),
        out_shape=jax.ShapeDtypeStruct((N, H), jnp.float32),
    )(h, part, w1, b1, w2, b2, lng, lnb, eps)


def kernel(x, edge_index, edge_attr,
           W_edge_0, b_edge_0, eps_0, W1_0, b1_0, bn1_g_0, bn1_b_0,
           W2_0, b2_0, bn_g_0, bn_b_0, ln_g_0, ln_b_0,
           W_edge_1, b_edge_1, eps_1, W1_1, b1_1, bn1_g_1, bn1_b_1,
           W2_1, b2_1, bn_g_1, bn_b_1, ln_g_1, ln_b_1,
           W_edge_2, b_edge_2, eps_2, W1_2, b1_2, bn1_g_2, bn1_b_2,
           W2_2, b2_2, bn_g_2, bn_b_2, ln_g_2, ln_b_2):
    bn_scale = 1.0 / jnp.sqrt(1.0 + 1e-5)
    pad_i = jnp.arange(E_PAD - E, dtype=jnp.int32)
    src_p = jnp.concatenate(
        [edge_index[0], pad_i % N]
    ).reshape(NC * NS, CHUNKS_PER_TILE, CHUNK)
    dst_p = jnp.concatenate(
        [edge_index[1], N + pad_i % (N_PAD - N)]
    ).reshape(NC * NS, CHUNKS_PER_TILE, CHUNK)
    sd_t = jnp.stack([src_p, dst_p], axis=0)
    zeros = jnp.zeros((N_PAD, H), jnp.float32)

    Ws, Es = [], []
    for (W_e, b_e, eps, W1, b1, g1, bb1, W2, b2, g2, bb2, lg, lb) in (
        (W_edge_0, b_edge_0, eps_0, W1_0, b1_0, bn1_g_0, bn1_b_0, W2_0, b2_0,
         bn_g_0, bn_b_0, ln_g_0, ln_b_0),
        (W_edge_1, b_edge_1, eps_1, W1_1, b1_1, bn1_g_1, bn1_b_1, W2_1, b2_1,
         bn_g_1, bn_b_1, ln_g_1, ln_b_1),
        (W_edge_2, b_edge_2, eps_2, W1_2, b1_2, bn1_g_2, bn1_b_2, W2_2, b2_2,
         bn_g_2, bn_b_2, ln_g_2, ln_b_2),
    ):
        s1 = bn_scale * g1
        w1f = W1 * s1[None, :]
        b1f = (b1 * s1 + bb1)[None, :]
        s2 = bn_scale * g2
        w2f = W2 * s2[None, :]
        b2f = (b2 * s2 + bb2)[None, :]
        Ws.append((eps.reshape(1), w1f, b1f, w2f, b2f,
                   lg[None, :], lb[None, :]))
        Es.append((W_e, b_e))

    h = x
    for i in range(3):
        eps, w1f, b1f, w2f, b2f, lg, lb = Ws[i]
        e = _edge_mlp(edge_attr, Es[i][0], Es[i][1][None, :])
        part = _sc_message_pass(h, e, sd_t, zeros)
        h = _node_mlp(h, part, w1f, b1f, w2f, b2f, lg, lb, eps,
                      residual=(i == 1))
    return h
